# baseline (speedup 1.0000x reference)
import functools

import jax
import jax.numpy as jnp
from jax import lax
from jax.experimental import pallas as pl
from jax.experimental.pallas import tpu as pltpu
from jax.experimental.pallas import tpu_sc as plsc

_B, _S, _D = 4, 2048, 768
_N = _B * _S
_NW = 32
_SPW = _S // _NW
_WR = 8
_NWIN = _SPW // _WR
_NQ = 3
_LANES = _D // 16


def _emb_body(idx_hbm, table_hbm, pos_hbm, out_hbm,
              idx_v, pos_v, *bufs_and_sems):
    qbuf = [[bufs_and_sems[s * _B + b] for b in range(_B)]
            for s in range(_NQ)]
    base = _NQ * _B
    gsem = [[bufs_and_sems[base + s * _B + b] for b in range(_B)]
            for s in range(_NQ)]
    base += _NQ * _B
    ssem = [[bufs_and_sems[base + s * _B + b] for b in range(_B)]
            for s in range(_NQ)]
    psem = bufs_and_sems[base + _NQ * _B]
    isem = bufs_and_sems[base + _NQ * _B + 1]
    nc = 2
    wid = lax.axis_index("s") * nc + lax.axis_index("c")
    pos0 = wid * _SPW

    pcp = pltpu.async_copy(pos_hbm.at[pl.ds(pos0, _SPW)], pos_v, psem)
    icp = [pltpu.async_copy(idx_hbm.at[pl.ds(b * _S + pos0, _SPW)],
                            idx_v.at[pl.ds(b * _SPW, _SPW)], isem)
           for b in range(_B)]
    for c in icp:
        c.wait()

    def idx_sl(w, b):
        return idx_v.at[pl.ds(b * _SPW + w * _WR, _WR)]

    def issue_gathers(w, s):
        for b in range(_B):
            pltpu.async_copy(table_hbm.at[idx_sl(w, b)], qbuf[s][b],
                             gsem[s][b])

    def wait_gathers(w, s):
        for b in range(_B):
            pltpu.make_async_copy(table_hbm.at[idx_sl(w, b)], qbuf[s][b],
                                  gsem[s][b]).wait()

    def out_ref(w, b):
        return out_hbm.at[pl.ds(b * _S + pos0 + w * _WR, _WR)]

    def issue_stores(w, s):
        for b in range(_B):
            pltpu.async_copy(qbuf[s][b], out_ref(w, b), ssem[s][b])

    def wait_stores(w, s):
        for b in range(_B):
            pltpu.make_async_copy(qbuf[s][b], out_ref(w, b),
                                  ssem[s][b]).wait()

    def add_rows(w, s):
        @plsc.parallel_loop(0, _WR)
        def row_body(r):
            for c in range(_LANES):
                sl = pl.ds(c * 16, 16)
                v = pos_v[w * _WR + r, sl]
                for b in range(_B):
                    plsc.addupdate(qbuf[s][b].at[r, sl], v)

    issue_gathers(0, 0)
    issue_gathers(1, 1)
    pcp.wait()

    for w in range(_NWIN):
        s = w % _NQ
        wait_gathers(w, s)
        add_rows(w, s)
        issue_stores(w, s)
        if w + 2 < _NWIN:
            if w >= 1:
                wait_stores(w - 1, (w + 2) % _NQ)
            issue_gathers(w + 2, (w + 2) % _NQ)

    for w in range(_NWIN - _NQ, _NWIN):
        wait_stores(w, w % _NQ)


@jax.jit
def _emb_lookup(idx_flat, token_table, pos_table):
    mesh = plsc.VectorSubcoreMesh(core_axis_name="c", subcore_axis_name="s")
    scratch = [
        pltpu.VMEM((_B * _SPW,), jnp.int32),
        pltpu.VMEM((_SPW, _D), jnp.float32),
    ]
    scratch += [pltpu.VMEM((_WR, _D), jnp.float32)
                for _ in range(_NQ * _B)]
    scratch += [pltpu.SemaphoreType.DMA
                for _ in range(2 * _NQ * _B)]
    scratch += [pltpu.SemaphoreType.DMA,
                pltpu.SemaphoreType.DMA]
    return pl.kernel(
        _emb_body,
        mesh=mesh,
        out_type=jax.ShapeDtypeStruct((_N, _D), jnp.float32),
        scratch_types=scratch,
    )(idx_flat, token_table, pos_table)


def kernel(embedding_idx, token_table, pos_table):
    b, s = embedding_idx.shape
    idx_flat = embedding_idx.reshape(b * s).astype(jnp.int32)
    out = _emb_lookup(idx_flat, token_table, pos_table)
    return out.reshape(b, s, token_table.shape[1])
